# Initial kernel scaffold; baseline (speedup 1.0000x reference)
import jax
import jax.numpy as jnp
from jax.experimental import pallas as pl


def kernel(x, edge_index, n_agents, carry, done_prev, W_gnn, b_gnn, ln_scale, ln_bias, Wi, Wh, bi, bh, W_g1, b_g1, w_g2, b_g2, W_h1, b_h1, w_o, b_o):
    raise NotImplementedError("write your pallas kernel here")


# speedup vs baseline: 5.4258x; 5.4258x over previous
import functools

import jax
import jax.numpy as jnp
from jax import lax
from jax.experimental import pallas as pl
from jax.experimental.pallas import tpu as pltpu
from jax.experimental.pallas import tpu_sc as plsc

_NC = 2
_NS = 16
_NW = _NC * _NS
_LANES = 16

_CHUNK = 80
_CP_CHUNKS = 5


def _sc_segment_mean_parts(x_aug, src, dst, n_pad):
    n, wa = x_aug.shape
    e = src.shape[0]
    assert wa % _LANES == 0
    assert e % (_NW * _CHUNK) == 0
    edges_per_tile = e // _NW
    n_chunks = edges_per_tile // _CHUNK
    rows_per_tile = n_pad // _NS
    assert rows_per_tile % (_CP_CHUNKS * 8) == 0
    cp_rows = rows_per_tile // _CP_CHUNKS

    mesh = plsc.VectorSubcoreMesh(
        core_axis_name="c", subcore_axis_name="s",
        num_cores=_NC, num_subcores=_NS)

    @functools.partial(
        pl.kernel,
        out_type=jax.ShapeDtypeStruct((_NC, n_pad, wa), jnp.float32),
        mesh=mesh,
        scratch_types=[
            pltpu.VMEM_SHARED((n_pad, wa), jnp.float32),
            pltpu.VMEM((cp_rows, wa), jnp.float32),
            pltpu.VMEM((_CHUNK, wa), jnp.float32),
            pltpu.VMEM((_CHUNK,), jnp.int32),
            pltpu.VMEM((_CHUNK,), jnp.int32),
            pltpu.SemaphoreType.DMA,
        ],
        compiler_params=pltpu.CompilerParams(use_tc_tiling_on_sc=False),
    )
    def seg(xaug_hbm, src_hbm, dst_hbm, out_hbm,
            accum, cbuf, rows_v, src_v, dst_v, sem):
        cid = lax.axis_index("c")
        sid = lax.axis_index("s")
        wid = sid * _NC + cid
        row_base = sid * rows_per_tile

        def zrow(i, c):
            for j in range(wa // _LANES):
                cbuf[i, pl.ds(j * _LANES, _LANES)] = (
                    jnp.zeros((_LANES,), jnp.float32))
            return c
        lax.fori_loop(0, cp_rows, zrow, 0)
        for t in range(_CP_CHUNKS):
            pltpu.sync_copy(
                cbuf, accum.at[pl.ds(row_base + t * cp_rows, cp_rows), :])
        plsc.subcore_barrier()

        e_base = wid * edges_per_tile

        def ebody(jc, c):
            eb = e_base + jc * _CHUNK
            pltpu.sync_copy(src_hbm.at[pl.ds(eb, _CHUNK)], src_v)
            pltpu.sync_copy(dst_hbm.at[pl.ds(eb, _CHUNK)], dst_v)
            pltpu.async_copy(xaug_hbm.at[src_v], rows_v, sem).wait()
            pltpu.sync_copy(rows_v, accum.at[dst_v], add=True)
            return c
        lax.fori_loop(0, n_chunks, ebody, 0)
        plsc.subcore_barrier()

        for t in range(_CP_CHUNKS):
            r0 = row_base + t * cp_rows
            pltpu.sync_copy(accum.at[pl.ds(r0, cp_rows), :], cbuf)
            pltpu.sync_copy(cbuf, out_hbm.at[cid, pl.ds(r0, cp_rows), :])

    return seg(x_aug, src, dst)


def _tc_dense(parts, carry, keep,
              W_gnn, b_gnn, ln_scale, ln_bias,
              Wi, Wh, bi, bh, W_g1, b_g1, w_g2, b_g2,
              W_h1, b_h1, w_o, b_o, *, n, d, hd, blk):
    nb = n // blk
    wa = parts.shape[2]

    def body(p0_ref, p1_ref, carry_ref, keep_ref,
             wg_ref, bg_ref, lns_ref, lnb_ref,
             wi_ref, wh_ref, bi_ref, bh_ref,
             wg1_ref, bg1_ref, wg2_ref, bg2_ref,
             wh1_ref, bh1_ref, wo_ref, bo_ref,
             val_ref, nc_ref,
             m_s, s_s, p_acc):
        i = pl.program_id(0)

        @pl.when(i == 0)
        def _init():
            m_s[0, 0] = -jnp.inf
            s_s[0, 0] = 0.0
            p_acc[...] = jnp.zeros_like(p_acc)

        p0 = p0_ref[0]
        p1 = p1_ref[0]
        agg = p0[:, :d] + p1[:, :d]
        degs = p0[:, d:] + p1[:, d:]
        deg = jnp.maximum(jnp.sum(degs, axis=-1, keepdims=True), 1.0)
        h = agg / deg
        h = jnp.maximum(
            jnp.dot(h, wg_ref[...], preferred_element_type=jnp.float32)
            + bg_ref[...], 0.0)
        mu = jnp.mean(h, axis=-1, keepdims=True)
        hc = h - mu
        var = jnp.mean(hc * hc, axis=-1, keepdims=True)
        h = hc / jnp.sqrt(var + 1e-6) * lns_ref[...] + lnb_ref[...]

        ce = carry_ref[...] * keep_ref[...]
        gx = jnp.dot(h, wi_ref[...], preferred_element_type=jnp.float32) \
            + bi_ref[...]
        gh = jnp.dot(ce, wh_ref[...], preferred_element_type=jnp.float32) \
            + bh_ref[...]
        r = jax.nn.sigmoid(gx[:, :hd] + gh[:, :hd])
        z = jax.nn.sigmoid(gx[:, hd:2 * hd] + gh[:, hd:2 * hd])
        nn = jnp.tanh(gx[:, 2 * hd:] + r * gh[:, 2 * hd:])
        nc = (1.0 - z) * nn + z * ce
        nc_ref[...] = nc

        gate = jnp.maximum(
            jnp.dot(nc, wg1_ref[...], preferred_element_type=jnp.float32)
            + bg1_ref[...], 0.0)
        gf = jnp.dot(gate, wg2_ref[...], preferred_element_type=jnp.float32) \
            + bg2_ref[...]

        bm = jnp.max(gf)
        m_old = m_s[0, 0]
        m_new = jnp.maximum(m_old, bm)
        alpha = jnp.exp(m_old - m_new)
        w = jnp.exp(gf - m_new)
        s_s[0, 0] = s_s[0, 0] * alpha + jnp.sum(w)
        p_acc[...] = p_acc[...] * alpha + jnp.sum(w * nc, axis=0,
                                                  keepdims=True)
        m_s[0, 0] = m_new

        @pl.when(i == nb - 1)
        def _final():
            pooled = p_acc[...] / s_s[0, 0]
            head = jnp.maximum(
                jnp.dot(pooled, wh1_ref[...],
                        preferred_element_type=jnp.float32) + bh1_ref[...],
                0.0)
            val_ref[...] = jnp.dot(
                head, wo_ref[...], preferred_element_type=jnp.float32) \
                + bo_ref[...]

    full = lambda a: pl.BlockSpec(a.shape, lambda i: (0,) * a.ndim)
    return pl.pallas_call(
        body,
        grid=(nb,),
        in_specs=[
            pl.BlockSpec((1, blk, wa), lambda i: (0, i, 0)),
            pl.BlockSpec((1, blk, wa), lambda i: (1, i, 0)),
            pl.BlockSpec((blk, hd), lambda i: (i, 0)),
            pl.BlockSpec((blk, 1), lambda i: (i, 0)),
            full(W_gnn), full(b_gnn), full(ln_scale), full(ln_bias),
            full(Wi), full(Wh), full(bi), full(bh),
            full(W_g1), full(b_g1), full(w_g2), full(b_g2),
            full(W_h1), full(b_h1), full(w_o), full(b_o),
        ],
        out_specs=[
            pl.BlockSpec((1, 1), lambda i: (0, 0)),
            pl.BlockSpec((blk, hd), lambda i: (i, 0)),
        ],
        out_shape=[
            jax.ShapeDtypeStruct((1, 1), jnp.float32),
            jax.ShapeDtypeStruct((n, hd), jnp.float32),
        ],
        scratch_shapes=[
            pltpu.SMEM((1, 1), jnp.float32),
            pltpu.SMEM((1, 1), jnp.float32),
            pltpu.VMEM((1, hd), jnp.float32),
        ],
    )(parts, parts, carry, keep,
      W_gnn, b_gnn, ln_scale, ln_bias, Wi, Wh, bi, bh,
      W_g1, b_g1, w_g2, b_g2, W_h1, b_h1, w_o, b_o)


def kernel(x, edge_index, n_agents, carry, done_prev,
           W_gnn, b_gnn, ln_scale, ln_bias, Wi, Wh, bi, bh,
           W_g1, b_g1, w_g2, b_g2, W_h1, b_h1, w_o, b_o):
    n, d = x.shape
    hd = W_gnn.shape[1]
    src = edge_index[0]
    dst = edge_index[1]

    x_aug = jnp.concatenate(
        [x, jnp.ones((n, 1), jnp.float32),
         jnp.zeros((n, _LANES - 1), jnp.float32)], axis=1)

    n_pad = 10240
    parts = _sc_segment_mean_parts(x_aug, src, dst, n_pad)

    keep = (1.0 - done_prev.astype(jnp.float32)).reshape(n, 1)
    val2, new_carry = _tc_dense(
        parts, carry, keep,
        W_gnn, b_gnn.reshape(1, -1), ln_scale.reshape(1, -1),
        ln_bias.reshape(1, -1), Wi, Wh, bi.reshape(1, -1),
        bh.reshape(1, -1), W_g1, b_g1.reshape(1, -1), w_g2,
        b_g2.reshape(1, -1), W_h1, b_h1.reshape(1, -1), w_o,
        b_o.reshape(1, -1), n=n, d=d, hd=hd, blk=400)
    return (val2.reshape(1), new_carry)
